# baseline (speedup 1.0000x reference)
import functools

import jax
import jax.numpy as jnp
from jax import lax
from jax.experimental import pallas as pl
from jax.experimental.pallas import tpu as pltpu
from jax.experimental.pallas import tpu_sc as plsc

B = 10000
K = 32
D = 128
ED = 128
NC, NS = 2, 16
BP = 10240
CHUNK = 128
RPC = CHUNK // K
TOTAL_CH = BP * K // CHUNK
CPT = TOTAL_CH // (NC * NS)
FLUSH_CH = 40
LANES = 16
DV = D // LANES
NBUF = 5

_mesh = plsc.VectorSubcoreMesh(core_axis_name="c", subcore_axis_name="s")


@functools.partial(
    pl.kernel,
    out_type=jax.ShapeDtypeStruct((BP, D), jnp.float32),
    mesh=_mesh,
    scratch_types=[
        pltpu.VMEM((CPT, CHUNK), jnp.int32),
        pltpu.VMEM((NBUF, CHUNK, D), jnp.float32),
        pltpu.VMEM((FLUSH_CH * RPC, D), jnp.float32),
        pltpu.SemaphoreType.DMA,
        pltpu.SemaphoreType.DMA,
        pltpu.SemaphoreType.DMA,
        pltpu.SemaphoreType.DMA,
        pltpu.SemaphoreType.DMA,
    ],
)
def _sc_gather_mean(nbr_hbm, table_hbm, agg_hbm, idx_v, rows_v, obuf,
                    sem0, sem1, sem2, sem3, sem4):
    cid = lax.axis_index("c")
    sid = lax.axis_index("s")
    sems = (sem0, sem1, sem2, sem3, sem4)

    def gather_start(c, slot):
        pltpu.async_copy(table_hbm.at[idx_v.at[c]], rows_v.at[slot], sems[slot])

    def gather_wait(slot):
        pltpu.make_async_copy(
            table_hbm.at[idx_v.at[0]], rows_v.at[slot], sems[slot]
        ).wait()

    def accum(c, local_c, slot):
        for r in range(RPC):
            def body(k, acc):
                row = r * K + k
                return tuple(
                    acc[d] + rows_v[slot, row, pl.ds(d * LANES, LANES)]
                    for d in range(DV)
                )
            acc = lax.fori_loop(
                0, K, body,
                tuple(jnp.zeros((LANES,), jnp.float32) for _ in range(DV)),
            )
            orow = local_c * RPC + r
            for d in range(DV):
                obuf[orow, pl.ds(d * LANES, LANES)] = acc[d] * (1.0 / K)

    def run(nch, cbase):
        with jax.named_scope("stage_idx"):
            pltpu.sync_copy(
                nbr_hbm.at[pl.ds(cbase, nch)], idx_v.at[pl.ds(0, nch)]
            )
        with jax.named_scope("prime"):
            for p in range(NBUF - 1):
                gather_start(p, p)

        for h in range(nch // FLUSH_CH):
            def outer(c0, carry):
                for b in range(NBUF):
                    c = h * FLUSH_CH + c0 * NBUF + b
                    gather_wait(b)

                    @pl.when(c + NBUF - 1 < nch)
                    def _():
                        gather_start(c + NBUF - 1, (b + NBUF - 1) % NBUF)

                    accum(c, c - h * FLUSH_CH, b)
                return carry

            with jax.named_scope("mainloop"):
                lax.fori_loop(0, FLUSH_CH // NBUF, outer, 0)
            with jax.named_scope("flush"):
                pltpu.sync_copy(
                    obuf,
                    agg_hbm.at[pl.ds((cbase + h * FLUSH_CH) * RPC,
                                     FLUSH_CH * RPC)],
                )

    run(CPT, (cid * NS + sid) * CPT)


def _tc_body(w_ref, a_ref, o_ref):
    o_ref[...] = jnp.maximum(
        lax.dot_general(
            w_ref[...], a_ref[...],
            dimension_numbers=(((1,), (1,)), ((), ())),
            preferred_element_type=jnp.float32,
        ),
        0.0,
    )


_BN = 1024

_tc_matmul = pl.pallas_call(
    _tc_body,
    grid=(BP // _BN,),
    in_specs=[
        pl.BlockSpec((ED, D), lambda i: (0, 0)),
        pl.BlockSpec((_BN, D), lambda i: (i, 0)),
    ],
    out_specs=pl.BlockSpec((ED, _BN), lambda i: (0, i)),
    out_shape=jax.ShapeDtypeStruct((ED, BP), jnp.float32),
)


def kernel(nodes, all_neighbors, feat_table, weight):
    del nodes
    nbr = all_neighbors.astype(jnp.int32)
    n_rows = feat_table.shape[0]
    pad = (jnp.arange((BP - B) * K, dtype=jnp.int32) % n_rows).reshape(BP - B, K)
    nbr = jnp.concatenate([nbr, pad], axis=0).reshape(TOTAL_CH, CHUNK)
    agg = _sc_gather_mean(nbr, feat_table)
    out = _tc_matmul(weight, agg)
    return out[:, :B]


# speedup vs baseline: 1.0260x; 1.0260x over previous
import functools

import jax
import jax.numpy as jnp
from jax import lax
from jax.experimental import pallas as pl
from jax.experimental.pallas import tpu as pltpu
from jax.experimental.pallas import tpu_sc as plsc

B = 10000
K = 32
D = 128
ED = 128
NC, NS = 2, 16
CHUNK = 128
RPC = CHUNK // K
BPAD = 10016
TOTAL_CH = BPAD * K // CHUNK
CPT = 80
LAST_CPT = TOTAL_CH - (NC * NS - 1) * CPT
MAXC = CPT
LANES = 16
DV = D // LANES
NBUF = 4

_mesh = plsc.VectorSubcoreMesh(core_axis_name="c", subcore_axis_name="s")


@functools.partial(
    pl.kernel,
    out_type=jax.ShapeDtypeStruct((BPAD, D), jnp.float32),
    mesh=_mesh,
    scratch_types=[
        pltpu.VMEM((MAXC, CHUNK), jnp.int32),
        pltpu.VMEM((NBUF, CHUNK, D), jnp.float32),
        pltpu.VMEM((MAXC * RPC, D), jnp.float32),
        pltpu.SemaphoreType.DMA,
        pltpu.SemaphoreType.DMA,
        pltpu.SemaphoreType.DMA,
        pltpu.SemaphoreType.DMA,
    ],
)
def _sc_gather_mean(nbr_hbm, table_hbm, agg_hbm, idx_v, rows_v, obuf,
                    sem0, sem1, sem2, sem3):
    w = lax.axis_index("c") * NS + lax.axis_index("s")
    cbase = w * CPT
    sems = (sem0, sem1, sem2, sem3)

    def gather_start(c, slot):
        pltpu.async_copy(table_hbm.at[idx_v.at[c]], rows_v.at[slot], sems[slot])

    def gather_wait(slot):
        pltpu.make_async_copy(
            table_hbm.at[idx_v.at[0]], rows_v.at[slot], sems[slot]
        ).wait()

    def accum(c, slot):
        for r in range(RPC):
            def body(k, acc):
                row = r * K + k
                return tuple(
                    acc[d] + rows_v[slot, row, pl.ds(d * LANES, LANES)]
                    for d in range(DV)
                )
            acc = lax.fori_loop(
                0, K, body,
                tuple(jnp.zeros((LANES,), jnp.float32) for _ in range(DV)),
            )
            orow = c * RPC + r
            for d in range(DV):
                obuf[orow, pl.ds(d * LANES, LANES)] = acc[d] * (1.0 / K)

    def run(nch):
        pltpu.sync_copy(
            nbr_hbm.at[pl.ds(cbase, nch)], idx_v.at[pl.ds(0, nch)]
        )
        for p in range(NBUF - 1):
            gather_start(p, p)

        def outer(c0, carry):
            for b in range(NBUF):
                c = c0 * NBUF + b
                gather_wait(b)

                @pl.when(c + NBUF - 1 < nch)
                def _():
                    gather_start(c + NBUF - 1, (b + NBUF - 1) % NBUF)

                accum(c, b)
            return carry

        main = nch - nch % NBUF
        lax.fori_loop(0, main // NBUF, outer, 0)
        for c in range(main, nch):
            gather_wait(c % NBUF)
            accum(c, c % NBUF)

        pltpu.sync_copy(
            obuf.at[pl.ds(0, nch * RPC)],
            agg_hbm.at[pl.ds(cbase * RPC, nch * RPC)],
        )

    @pl.when(w < NC * NS - 1)
    def _():
        run(CPT)

    @pl.when(w == NC * NS - 1)
    def _():
        run(LAST_CPT)


def _tc_body(w_ref, a_ref, o_ref):
    o_ref[...] = jnp.maximum(
        lax.dot_general(
            w_ref[...], a_ref[...],
            dimension_numbers=(((1,), (1,)), ((), ())),
            preferred_element_type=jnp.float32,
        ),
        0.0,
    )


_BN = 1024

_tc_matmul = pl.pallas_call(
    _tc_body,
    grid=(pl.cdiv(B, _BN),),
    in_specs=[
        pl.BlockSpec((ED, D), lambda i: (0, 0)),
        pl.BlockSpec((_BN, D), lambda i: (i, 0)),
    ],
    out_specs=pl.BlockSpec((ED, _BN), lambda i: (0, i)),
    out_shape=jax.ShapeDtypeStruct((ED, B), jnp.float32),
)


def kernel(nodes, all_neighbors, feat_table, weight):
    del nodes
    nbr = all_neighbors.astype(jnp.int32)
    n_rows = feat_table.shape[0]
    pad = (jnp.arange((BPAD - B) * K, dtype=jnp.int32) % n_rows).reshape(
        BPAD - B, K)
    nbr = jnp.concatenate([nbr, pad], axis=0).reshape(TOTAL_CH, CHUNK)
    agg = _sc_gather_mean(nbr, feat_table)
    return _tc_matmul(weight, agg)


# speedup vs baseline: 1.0467x; 1.0202x over previous
import functools

import jax
import jax.numpy as jnp
from jax import lax
from jax.experimental import pallas as pl
from jax.experimental.pallas import tpu as pltpu
from jax.experimental.pallas import tpu_sc as plsc

B = 10000
K = 32
D = 128
ED = 128
NC, NS = 2, 16
CHUNK = 128
RPC = CHUNK // K
BPAD = 10016
TOTAL_CH = BPAD * K // CHUNK
CPT = 80
LAST_CPT = TOTAL_CH - (NC * NS - 1) * CPT
MAXC = CPT
LANES = 16
DV = D // LANES
NBUF = 4

_mesh = plsc.VectorSubcoreMesh(core_axis_name="c", subcore_axis_name="s")


@functools.partial(
    pl.kernel,
    out_type=jax.ShapeDtypeStruct((BPAD, D), jnp.float32),
    mesh=_mesh,
    scratch_types=[
        pltpu.VMEM((MAXC, CHUNK), jnp.int32),
        pltpu.VMEM((NBUF, CHUNK, D), jnp.float32),
        pltpu.VMEM((MAXC * RPC, D), jnp.float32),
        pltpu.SemaphoreType.DMA,
        pltpu.SemaphoreType.DMA,
        pltpu.SemaphoreType.DMA,
        pltpu.SemaphoreType.DMA,
    ],
)
def _sc_gather_mean(nbr_hbm, table_hbm, agg_hbm, idx_v, rows_v, obuf,
                    sem0, sem1, sem2, sem3):
    w = lax.axis_index("c") * NS + lax.axis_index("s")
    cbase = w * CPT
    sems = (sem0, sem1, sem2, sem3)

    def gather_start(c, slot):
        pltpu.async_copy(table_hbm.at[idx_v.at[c]], rows_v.at[slot], sems[slot])

    def gather_wait(slot):
        pltpu.make_async_copy(
            table_hbm.at[idx_v.at[0]], rows_v.at[slot], sems[slot]
        ).wait()

    def accum(c, slot):
        for r in range(RPC):
            def body(k, acc):
                row = r * K + k
                return tuple(
                    acc[d] + rows_v[slot, row, pl.ds(d * LANES, LANES)]
                    for d in range(DV)
                )
            acc = lax.fori_loop(
                0, K, body,
                tuple(jnp.zeros((LANES,), jnp.float32) for _ in range(DV)),
            )
            orow = c * RPC + r
            for d in range(DV):
                obuf[orow, pl.ds(d * LANES, LANES)] = acc[d] * (1.0 / K)

    def run(nch):
        pltpu.sync_copy(
            nbr_hbm.at[pl.ds(cbase, nch)], idx_v.at[pl.ds(0, nch)]
        )
        for p in range(NBUF - 1):
            gather_start(p, p)

        def outer(c0, carry):
            for b in range(NBUF):
                c = c0 * NBUF + b
                gather_wait(b)

                @pl.when(c + NBUF - 1 < nch)
                def _():
                    gather_start(c + NBUF - 1, (b + NBUF - 1) % NBUF)

                accum(c, b)
            return carry

        main = nch - nch % NBUF
        lax.fori_loop(0, main // NBUF, outer, 0)
        for c in range(main, nch):
            gather_wait(c % NBUF)
            accum(c, c % NBUF)

        pltpu.sync_copy(
            obuf.at[pl.ds(0, nch * RPC)],
            agg_hbm.at[pl.ds(cbase * RPC, nch * RPC)],
        )

    @pl.when(w < NC * NS - 1)
    def _():
        run(CPT)

    @pl.when(w == NC * NS - 1)
    def _():
        run(LAST_CPT)


def _tc_body(w_ref, a_ref, o_ref):
    o_ref[...] = jnp.maximum(
        lax.dot_general(
            w_ref[...], a_ref[...],
            dimension_numbers=(((1,), (1,)), ((), ())),
            preferred_element_type=jnp.float32,
        ),
        0.0,
    )


_BN = 2048

_tc_matmul = pl.pallas_call(
    _tc_body,
    grid=(pl.cdiv(B, _BN),),
    in_specs=[
        pl.BlockSpec((ED, D), lambda i: (0, 0)),
        pl.BlockSpec((_BN, D), lambda i: (i, 0)),
    ],
    out_specs=pl.BlockSpec((ED, _BN), lambda i: (0, i)),
    out_shape=jax.ShapeDtypeStruct((ED, B), jnp.float32),
)


def kernel(nodes, all_neighbors, feat_table, weight):
    del nodes
    nbr = all_neighbors.astype(jnp.int32)
    n_rows = feat_table.shape[0]
    pad = (jnp.arange((BPAD - B) * K, dtype=jnp.int32) % n_rows).reshape(
        BPAD - B, K)
    nbr = jnp.concatenate([nbr, pad], axis=0).reshape(TOTAL_CH, CHUNK)
    agg = _sc_gather_mean(nbr, feat_table)
    return _tc_matmul(weight, agg)
